# Initial kernel scaffold; baseline (speedup 1.0000x reference)
import jax
import jax.numpy as jnp
from jax.experimental import pallas as pl


def kernel(x1, x2, embedding, fc_w, fc_b):
    raise NotImplementedError("write your pallas kernel here")


# speedup vs baseline: 7.6212x; 7.6212x over previous
import functools

import jax
import jax.numpy as jnp
from jax import lax
from jax.experimental import pallas as pl
from jax.experimental.pallas import tpu as pltpu
from jax.experimental.pallas import tpu_sc as plsc

VOCAB = 100000
EMBED = 128
BATCH = 4096
SEQ = 50

NC = 2
NS = 16
NW = NC * NS
BPW = BATCH // NW
L = 16
NCH = EMBED // L

_mesh = plsc.VectorSubcoreMesh(core_axis_name="c", subcore_axis_name="s")


def _sc_body(x1_hbm, x2_hbm, emb_hbm, q1_hbm, q2_hbm,
             idx1_v, idx2_v, rows1_v, rows2_v, res1_v, res2_v, sem1, sem2):
  wid = lax.axis_index("s") * NC + lax.axis_index("c")
  base = wid * BPW

  pltpu.sync_copy(x1_hbm.at[pl.ds(base, BPW)], idx1_v)
  pltpu.sync_copy(x2_hbm.at[pl.ds(base, BPW)], idx2_v)

  def elem(b, carry):
    cp1 = pltpu.async_copy(emb_hbm.at[idx1_v.at[b]], rows1_v, sem1)
    cp2 = pltpu.async_copy(emb_hbm.at[idx2_v.at[b]], rows2_v, sem2)
    cp1.wait()
    cp2.wait()

    def rbody(r, accs):
      new = []
      for c in range(NCH):
        new.append(accs[c] + rows1_v[r, pl.ds(c * L, L)])
      for c in range(NCH):
        new.append(accs[NCH + c] + rows2_v[r, pl.ds(c * L, L)])
      return tuple(new)

    init = tuple(
        [rows1_v[0, pl.ds(c * L, L)] for c in range(NCH)]
        + [rows2_v[0, pl.ds(c * L, L)] for c in range(NCH)])
    accs = lax.fori_loop(1, SEQ, rbody, init)
    for c in range(NCH):
      res1_v[b, pl.ds(c * L, L)] = accs[c]
      res2_v[b, pl.ds(c * L, L)] = accs[NCH + c]
    return carry

  lax.fori_loop(0, BPW, elem, 0)

  pltpu.sync_copy(res1_v, q1_hbm.at[pl.ds(base, BPW)])
  pltpu.sync_copy(res2_v, q2_hbm.at[pl.ds(base, BPW)])


_sc_pool = functools.partial(
    pl.kernel,
    out_type=(
        jax.ShapeDtypeStruct((BATCH, EMBED), jnp.float32),
        jax.ShapeDtypeStruct((BATCH, EMBED), jnp.float32),
    ),
    mesh=_mesh,
    scratch_types=[
        pltpu.VMEM((BPW, SEQ), jnp.int32),
        pltpu.VMEM((BPW, SEQ), jnp.int32),
        pltpu.VMEM((SEQ, EMBED), jnp.float32),
        pltpu.VMEM((SEQ, EMBED), jnp.float32),
        pltpu.VMEM((BPW, EMBED), jnp.float32),
        pltpu.VMEM((BPW, EMBED), jnp.float32),
        pltpu.SemaphoreType.DMA,
        pltpu.SemaphoreType.DMA,
    ],
)(_sc_body)


def _tc_body(q1_ref, q2_ref, w_ref, b_ref, out_ref):
  q1 = q1_ref[...] * (1.0 / SEQ)
  q2 = q2_ref[...] * (1.0 / SEQ)
  eps = 1e-8
  n1 = jnp.maximum(jnp.sqrt(jnp.sum(q1 * q1, axis=1, keepdims=True)), eps)
  n2 = jnp.maximum(jnp.sqrt(jnp.sum(q2 * q2, axis=1, keepdims=True)), eps)
  cos = jnp.sum(q1 * q2, axis=1, keepdims=True) / (n1 * n2)
  out_ref[...] = cos * w_ref[...] + b_ref[...]


def _tc_epilogue(q1s, q2s, w_t, b_t):
  return pl.pallas_call(
      _tc_body,
      out_shape=jax.ShapeDtypeStruct((BATCH, 2), jnp.float32),
  )(q1s, q2s, w_t, b_t)


def kernel(x1, x2, embedding, fc_w, fc_b):
  x1 = x1.astype(jnp.int32)
  x2 = x2.astype(jnp.int32)
  q1s, q2s = _sc_pool(x1, x2, embedding)
  w_t = fc_w.reshape(1, 2)
  b_t = fc_b.reshape(1, 2)
  return _tc_epilogue(q1s, q2s, w_t, b_t)


# speedup vs baseline: 11.9909x; 1.5734x over previous
import functools

import jax
import jax.numpy as jnp
from jax import lax
from jax.experimental import pallas as pl
from jax.experimental.pallas import tpu as pltpu
from jax.experimental.pallas import tpu_sc as plsc

VOCAB = 100000
EMBED = 128
BATCH = 4096
SEQ = 50

NC = 2
NS = 16
NW = NC * NS
BPW = BATCH // NW
L = 16
NCH = EMBED // L

_mesh = plsc.VectorSubcoreMesh(core_axis_name="c", subcore_axis_name="s")


def _sc_body(xcat_hbm, emb_hbm, q1_hbm, q2_hbm,
             idx_v, rowsa_v, rowsb_v, res1_v, res2_v, sema, semb):
  wid = lax.axis_index("s") * NC + lax.axis_index("c")
  base = wid * BPW

  pltpu.sync_copy(xcat_hbm.at[pl.ds(base, BPW)], idx_v)

  def gather(b, rows_v, sem):
    return pltpu.make_async_copy(emb_hbm.at[idx_v.at[b]], rows_v, sem)

  def accumulate(rows_v, b):
    def rbody(r, accs):
      new = []
      for c in range(NCH):
        new.append(accs[c] + rows_v[r, pl.ds(c * L, L)])
      for c in range(NCH):
        new.append(accs[NCH + c] + rows_v[SEQ + r, pl.ds(c * L, L)])
      return tuple(new)

    init = tuple(
        [rows_v[0, pl.ds(c * L, L)] for c in range(NCH)]
        + [rows_v[SEQ, pl.ds(c * L, L)] for c in range(NCH)])
    accs = lax.fori_loop(1, SEQ, rbody, init, unroll=2)
    for c in range(NCH):
      res1_v[b, pl.ds(c * L, L)] = accs[c]
      res2_v[b, pl.ds(c * L, L)] = accs[NCH + c]

  gather(0, rowsa_v, sema).start()
  gather(1, rowsb_v, semb).start()

  def pair(i, carry):
    b0 = 2 * i
    gather(b0, rowsa_v, sema).wait()
    accumulate(rowsa_v, b0)

    @pl.when(b0 + 2 < BPW)
    def _():
      gather(b0 + 2, rowsa_v, sema).start()

    b1 = b0 + 1
    gather(b1, rowsb_v, semb).wait()
    accumulate(rowsb_v, b1)

    @pl.when(b1 + 2 < BPW)
    def _():
      gather(b1 + 2, rowsb_v, semb).start()

    return carry

  lax.fori_loop(0, BPW // 2, pair, 0)

  pltpu.sync_copy(res1_v, q1_hbm.at[pl.ds(base, BPW)])
  pltpu.sync_copy(res2_v, q2_hbm.at[pl.ds(base, BPW)])


_sc_pool = functools.partial(
    pl.kernel,
    out_type=(
        jax.ShapeDtypeStruct((BATCH, EMBED), jnp.float32),
        jax.ShapeDtypeStruct((BATCH, EMBED), jnp.float32),
    ),
    mesh=_mesh,
    scratch_types=[
        pltpu.VMEM((BPW, 2 * SEQ), jnp.int32),
        pltpu.VMEM((2 * SEQ, EMBED), jnp.float32),
        pltpu.VMEM((2 * SEQ, EMBED), jnp.float32),
        pltpu.VMEM((BPW, EMBED), jnp.float32),
        pltpu.VMEM((BPW, EMBED), jnp.float32),
        pltpu.SemaphoreType.DMA,
        pltpu.SemaphoreType.DMA,
    ],
)(_sc_body)


def _tc_body(q1_ref, q2_ref, w_ref, b_ref, out_ref):
  q1 = q1_ref[...] * (1.0 / SEQ)
  q2 = q2_ref[...] * (1.0 / SEQ)
  eps = 1e-8
  n1 = jnp.maximum(jnp.sqrt(jnp.sum(q1 * q1, axis=1, keepdims=True)), eps)
  n2 = jnp.maximum(jnp.sqrt(jnp.sum(q2 * q2, axis=1, keepdims=True)), eps)
  cos = jnp.sum(q1 * q2, axis=1, keepdims=True) / (n1 * n2)
  out_ref[...] = cos * w_ref[...] + b_ref[...]


def _tc_epilogue(q1s, q2s, w_t, b_t):
  return pl.pallas_call(
      _tc_body,
      out_shape=jax.ShapeDtypeStruct((BATCH, 2), jnp.float32),
  )(q1s, q2s, w_t, b_t)


def kernel(x1, x2, embedding, fc_w, fc_b):
  xcat = jnp.concatenate(
      [x1.astype(jnp.int32), x2.astype(jnp.int32)], axis=1)
  q1s, q2s = _sc_pool(xcat, embedding)
  w_t = fc_w.reshape(1, 2)
  b_t = fc_b.reshape(1, 2)
  return _tc_epilogue(q1s, q2s, w_t, b_t)


# speedup vs baseline: 15.1361x; 1.2623x over previous
import functools

import jax
import jax.numpy as jnp
from jax import lax
from jax.experimental import pallas as pl
from jax.experimental.pallas import tpu as pltpu
from jax.experimental.pallas import tpu_sc as plsc

VOCAB = 100000
EMBED = 128
BATCH = 4096
SEQ = 50

NC = 2
NS = 16
NW = NC * NS
BPW = BATCH // NW
L = 16
NCH = EMBED // L
NBUF = 4

_mesh = plsc.VectorSubcoreMesh(core_axis_name="c", subcore_axis_name="s")


def _sc_body(xcat_hbm, emb_hbm, q1_hbm, q2_hbm,
             idx_v, rows0_v, rows1_v, rows2_v, rows3_v, res1_v, res2_v,
             sem0, sem1, sem2, sem3):
  rows = [rows0_v, rows1_v, rows2_v, rows3_v]
  sems = [sem0, sem1, sem2, sem3]

  wid = lax.axis_index("s") * NC + lax.axis_index("c")
  base = wid * BPW

  pltpu.sync_copy(xcat_hbm.at[pl.ds(base, BPW)], idx_v)

  def gather(b, j):
    return pltpu.make_async_copy(emb_hbm.at[idx_v.at[b]], rows[j], sems[j])

  def accumulate(rows_v, b):
    def rbody(r, accs):
      new = []
      for c in range(NCH):
        new.append(accs[c] + rows_v[r, pl.ds(c * L, L)])
      for c in range(NCH):
        new.append(accs[NCH + c] + rows_v[SEQ + r, pl.ds(c * L, L)])
      return tuple(new)

    init = tuple(
        [rows_v[0, pl.ds(c * L, L)] for c in range(NCH)]
        + [rows_v[SEQ, pl.ds(c * L, L)] for c in range(NCH)])
    accs = lax.fori_loop(1, SEQ, rbody, init, unroll=2)
    for c in range(NCH):
      res1_v[b, pl.ds(c * L, L)] = accs[c]
      res2_v[b, pl.ds(c * L, L)] = accs[NCH + c]

  for j in range(NBUF - 1):
    gather(j, j).start()

  def group(i, carry):
    b0 = NBUF * i
    for j in range(NBUF):
      b = b0 + j
      gather(b, j).wait()
      accumulate(rows[j], b)

      @pl.when(b + NBUF - 1 < BPW)
      def _():
        gather(b + NBUF - 1, (j + NBUF - 1) % NBUF).start()

    return carry

  lax.fori_loop(0, BPW // NBUF, group, 0)

  pltpu.sync_copy(res1_v, q1_hbm.at[pl.ds(base, BPW)])
  pltpu.sync_copy(res2_v, q2_hbm.at[pl.ds(base, BPW)])


_sc_pool = functools.partial(
    pl.kernel,
    out_type=(
        jax.ShapeDtypeStruct((BATCH, EMBED), jnp.float32),
        jax.ShapeDtypeStruct((BATCH, EMBED), jnp.float32),
    ),
    mesh=_mesh,
    scratch_types=[
        pltpu.VMEM((BPW, 2 * SEQ), jnp.int32),
        pltpu.VMEM((2 * SEQ, EMBED), jnp.float32),
        pltpu.VMEM((2 * SEQ, EMBED), jnp.float32),
        pltpu.VMEM((2 * SEQ, EMBED), jnp.float32),
        pltpu.VMEM((2 * SEQ, EMBED), jnp.float32),
        pltpu.VMEM((BPW, EMBED), jnp.float32),
        pltpu.VMEM((BPW, EMBED), jnp.float32),
        pltpu.SemaphoreType.DMA,
        pltpu.SemaphoreType.DMA,
        pltpu.SemaphoreType.DMA,
        pltpu.SemaphoreType.DMA,
    ],
)(_sc_body)


def _tc_body(q1_ref, q2_ref, w_ref, b_ref, out_ref):
  q1 = q1_ref[...] * (1.0 / SEQ)
  q2 = q2_ref[...] * (1.0 / SEQ)
  eps = 1e-8
  n1 = jnp.maximum(jnp.sqrt(jnp.sum(q1 * q1, axis=1, keepdims=True)), eps)
  n2 = jnp.maximum(jnp.sqrt(jnp.sum(q2 * q2, axis=1, keepdims=True)), eps)
  cos = jnp.sum(q1 * q2, axis=1, keepdims=True) / (n1 * n2)
  out_ref[...] = cos * w_ref[...] + b_ref[...]


def _tc_epilogue(q1s, q2s, w_t, b_t):
  return pl.pallas_call(
      _tc_body,
      out_shape=jax.ShapeDtypeStruct((BATCH, 2), jnp.float32),
  )(q1s, q2s, w_t, b_t)


def kernel(x1, x2, embedding, fc_w, fc_b):
  xcat = jnp.concatenate(
      [x1.astype(jnp.int32), x2.astype(jnp.int32)], axis=1)
  q1s, q2s = _sc_pool(xcat, embedding)
  w_t = fc_w.reshape(1, 2)
  b_t = fc_b.reshape(1, 2)
  return _tc_epilogue(q1s, q2s, w_t, b_t)


# speedup vs baseline: 16.6409x; 1.0994x over previous
import functools

import jax
import jax.numpy as jnp
from jax import lax
from jax.experimental import pallas as pl
from jax.experimental.pallas import tpu as pltpu
from jax.experimental.pallas import tpu_sc as plsc

VOCAB = 100000
EMBED = 128
BATCH = 4096
SEQ = 50

NC = 2
NS = 16
NW = NC * NS
BPW = BATCH // NW
L = 16
NCH = EMBED // L
NBUF = 6

_mesh = plsc.VectorSubcoreMesh(core_axis_name="c", subcore_axis_name="s")


def _sc_body(xcat_hbm, emb_hbm, part_hbm, *refs):
  rows = list(refs[1:1 + NBUF])
  sems = list(refs[2 + NBUF:2 + 2 * NBUF])
  idx_v = refs[0]
  part_v = refs[1 + NBUF]

  wid = lax.axis_index("s") * NC + lax.axis_index("c")
  base = wid * BPW

  pltpu.sync_copy(xcat_hbm.at[pl.ds(base, BPW)], idx_v)

  def gather(b, j):
    return pltpu.make_async_copy(emb_hbm.at[idx_v.at[b]], rows[j], sems[j])

  def accumulate(rows_v, b):
    def rbody(r, accs):
      new = []
      for c in range(NCH):
        new.append(accs[c] + rows_v[r, pl.ds(c * L, L)])
      for c in range(NCH):
        new.append(accs[NCH + c] + rows_v[SEQ + r, pl.ds(c * L, L)])
      return tuple(new)

    init = tuple(
        [rows_v[0, pl.ds(c * L, L)] for c in range(NCH)]
        + [rows_v[SEQ, pl.ds(c * L, L)] for c in range(NCH)])
    accs = lax.fori_loop(1, SEQ, rbody, init, unroll=2)
    dotv = accs[0] * accs[NCH]
    ss1v = accs[0] * accs[0]
    ss2v = accs[NCH] * accs[NCH]
    for c in range(1, NCH):
      dotv += accs[c] * accs[NCH + c]
      ss1v += accs[c] * accs[c]
      ss2v += accs[NCH + c] * accs[NCH + c]
    part_v[b, pl.ds(0, L)] = dotv
    part_v[b, pl.ds(L, L)] = ss1v
    part_v[b, pl.ds(2 * L, L)] = ss2v

  for j in range(NBUF - 1):
    gather(j, j).start()

  def group(i, carry):
    b0 = NBUF * i
    for j in range(NBUF):
      b = b0 + j
      gather(b, j).wait()
      accumulate(rows[j], b)

      @pl.when(b + NBUF - 1 < BPW)
      def _():
        gather(b + NBUF - 1, (j + NBUF - 1) % NBUF).start()

    return carry

  n_groups = BPW // NBUF
  lax.fori_loop(0, n_groups, group, 0)
  for b in range(n_groups * NBUF, BPW):
    gather(b, b % NBUF).wait()
    accumulate(rows[b % NBUF], b)

  pltpu.sync_copy(part_v, part_hbm.at[pl.ds(base, BPW)])


_sc_pool = functools.partial(
    pl.kernel,
    out_type=jax.ShapeDtypeStruct((BATCH, 3 * L), jnp.float32),
    mesh=_mesh,
    scratch_types=(
        [pltpu.VMEM((BPW, 2 * SEQ), jnp.int32)]
        + [pltpu.VMEM((2 * SEQ, EMBED), jnp.float32) for _ in range(NBUF)]
        + [pltpu.VMEM((BPW, 3 * L), jnp.float32)]
        + [pltpu.SemaphoreType.DMA for _ in range(NBUF)]
    ),
)(_sc_body)


def _tc_body(part_ref, w_ref, b_ref, out_ref):
  part = part_ref[...]
  dot = jnp.sum(part[:, 0:L], axis=1, keepdims=True) * (1.0 / (SEQ * SEQ))
  ss1 = jnp.sum(part[:, L:2 * L], axis=1, keepdims=True)
  ss2 = jnp.sum(part[:, 2 * L:3 * L], axis=1, keepdims=True)
  eps = 1e-8
  n1 = jnp.maximum(jnp.sqrt(ss1) * (1.0 / SEQ), eps)
  n2 = jnp.maximum(jnp.sqrt(ss2) * (1.0 / SEQ), eps)
  cos = dot / (n1 * n2)
  out_ref[...] = cos * w_ref[...] + b_ref[...]


def _tc_epilogue(part, w_t, b_t):
  return pl.pallas_call(
      _tc_body,
      out_shape=jax.ShapeDtypeStruct((BATCH, 2), jnp.float32),
  )(part, w_t, b_t)


def kernel(x1, x2, embedding, fc_w, fc_b):
  xcat = jnp.concatenate(
      [x1.astype(jnp.int32), x2.astype(jnp.int32)], axis=1)
  part = _sc_pool(xcat, embedding)
  w_t = fc_w.reshape(1, 2)
  b_t = fc_b.reshape(1, 2)
  return _tc_epilogue(part, w_t, b_t)
